# Initial kernel scaffold; baseline (speedup 1.0000x reference)
import jax
import jax.numpy as jnp
from jax.experimental import pallas as pl


def kernel(h0, edge_index, W1, a1s, a1d, b1, W2, a2s, a2d, b2, W3, a3s, a3d, b3, W4, a4s, a4d, b4, W5, a5s, a5d, b5, Wl):
    raise NotImplementedError("write your pallas kernel here")


# speedup vs baseline: 19.2812x; 19.2812x over previous
import functools

import jax
import jax.numpy as jnp
import numpy as np
from jax import lax
from jax.experimental import pallas as pl
from jax.experimental.pallas import tpu as pltpu
from jax.experimental.pallas import tpu_sc as plsc

N = 10000
IN = 128
HID = 16
H = 8
OUT = 128
EMB = 64

NC = 2
NS = 16
CHUNK = 32
E1 = 320000 + N
CPT = -(-E1 // (NS * CHUNK))
EPAD = NS * CHUNK * CPT
NPAD = 10240
RPT = NPAD // NS
PCH = RPT // 16


@functools.partial(
    pl.kernel,
    out_type=[
        jax.ShapeDtypeStruct((NC, NPAD, 128), jnp.float32),
    ],
    mesh=plsc.VectorSubcoreMesh(core_axis_name="c", subcore_axis_name="s"),
    scratch_types=[
        pltpu.VMEM((CHUNK,), jnp.int32),
        pltpu.VMEM((CHUNK,), jnp.int32),
        pltpu.VMEM((CHUNK, 128), jnp.float32),
        pltpu.VMEM((CHUNK, 128), jnp.float32),
        pltpu.VMEM((CHUNK, 128), jnp.float32),
        pltpu.VMEM((PCH, 128), jnp.float32),
        pltpu.VMEM_SHARED((NPAD, 128), jnp.float32),
        pltpu.SemaphoreType.DMA,
        pltpu.SemaphoreType.DMA,
    ],
)
def _gat_edges_sc(g0_hbm, g1_hbm, d_hbm, srcp_hbm, dstp_hbm, z_hbm,
                  packed_hbm,
                  src_v, dst_v, g_v, d_v, x_v, nb_v, nacc, sem, sem2):
    cid = lax.axis_index("c")
    sid = lax.axis_index("s")
    r0 = sid * RPT
    pltpu.sync_copy(z_hbm, nb_v)
    for i in range(RPT // PCH):
        rr = r0 + i * PCH
        pltpu.sync_copy(nb_v, nacc.at[pl.ds(rr, PCH)])

    def zrow(e, c):
        for j in range(3):
            x_v[e, pl.ds(80 + 16 * j, 16)] = jnp.zeros((16,), jnp.float32)
        return c

    lax.fori_loop(0, CHUNK, zrow, 0)
    plsc.subcore_barrier()

    def run(g_hbm, doff):
        base_t = sid * (CPT * CHUNK)

        def chunk_body(k, carry):
            base = base_t + k * CHUNK
            pltpu.sync_copy(srcp_hbm.at[pl.ds(base, CHUNK)], src_v)
            pltpu.sync_copy(dstp_hbm.at[pl.ds(base, CHUNK)], dst_v)
            pltpu.async_copy(g_hbm.at[src_v], g_v, sem).wait()
            pltpu.async_copy(d_hbm.at[dst_v], d_v, sem2).wait()

            lane4 = lax.iota(jnp.int32, 16) & 3

            def edge_body(e, c2):
                ws = []
                for j in range(4):
                    t = (g_v[e, pl.ds(64 + 16 * j, 16)]
                         + d_v[e, pl.ds(doff + 16 * j, 16)])
                    t = jnp.where(t > 0.0, t, 0.2 * t)
                    w = jnp.exp(t)
                    ws.append(w)
                    x_v[e, pl.ds(16 * j, 16)] = g_v[e, pl.ds(16 * j, 16)] * w
                dv = ws[3]
                for j in range(3):
                    dv = jnp.where(lane4 == j, ws[j], dv)
                x_v[e, pl.ds(64, 16)] = dv
                return c2

            lax.fori_loop(0, CHUNK, edge_body, 0)
            pltpu.sync_copy(x_v, nacc.at[dst_v], add=True)
            return carry

        lax.fori_loop(0, CPT, chunk_body, 0)

    @pl.when(cid == 0)
    def _():
        run(g0_hbm, 0)

    @pl.when(cid == 1)
    def _():
        run(g1_hbm, 64)

    plsc.subcore_barrier()
    for i in range(RPT // PCH):
        rr = r0 + i * PCH
        pltpu.sync_copy(nacc.at[pl.ds(rr, PCH)], nb_v)
        pltpu.sync_copy(nb_v, packed_hbm.at[cid, pl.ds(rr, PCH)])


def _emit_tables(hcur, m0_ref, m1_ref, md_ref, g0_ref, g1_ref, d_ref):
    g0_ref[...] = jnp.dot(hcur, m0_ref[...], preferred_element_type=jnp.float32)
    g1_ref[...] = jnp.dot(hcur, m1_ref[...], preferred_element_type=jnp.float32)
    d_ref[N:NPAD, :] = jnp.zeros((NPAD - N, 128), jnp.float32)
    d_ref[0:N, :] = jnp.dot(hcur, md_ref[...], preferred_element_type=jnp.float32)


_TABLES_OUT = [
    jax.ShapeDtypeStruct((N, 128), jnp.float32),
    jax.ShapeDtypeStruct((N, 128), jnp.float32),
    jax.ShapeDtypeStruct((NPAD, 128), jnp.float32),
]


def _tc_first(h0, M0, M1, MD):
    def body(h_ref, m0_ref, m1_ref, md_ref, g0_ref, g1_ref, d_ref):
        _emit_tables(h_ref[...], m0_ref, m1_ref, md_ref, g0_ref, g1_ref, d_ref)

    return pl.pallas_call(body, out_shape=_TABLES_OUT)(h0, M0, M1, MD)


def _norm_prev(pk_ref, b_ref, rme_ref):
    numer = jnp.concatenate([pk_ref[0, 0:N, 0:64], pk_ref[1, 0:N, 0:64]],
                            axis=1)
    rme = rme_ref[...]
    denom = jnp.concatenate(
        [jnp.dot(pk_ref[0, 0:N, 64:80], rme,
                 preferred_element_type=jnp.float32),
         jnp.dot(pk_ref[1, 0:N, 64:80], rme,
                 preferred_element_type=jnp.float32)],
        axis=1)
    return jnp.maximum(numer / (denom + 1e-16) + b_ref[...], 0.0)


def _tc_mid(packed_p, b2d, rme, M0, M1, MD):
    def body(pk_ref, b_ref, rme_ref, m0_ref, m1_ref, md_ref,
             g0_ref, g1_ref, d_ref):
        hcur = _norm_prev(pk_ref, b_ref, rme_ref)
        _emit_tables(hcur, m0_ref, m1_ref, md_ref, g0_ref, g1_ref, d_ref)

    return pl.pallas_call(body, out_shape=_TABLES_OUT)(
        packed_p, b2d, rme, M0, M1, MD)


def _tc_final(packed_p, b2d, rme, h0, Wlt):
    def body(pk_ref, b_ref, rme_ref, h0_ref, wlt_ref, out_ref):
        h5 = _norm_prev(pk_ref, b_ref, rme_ref)
        tot = jnp.sum(h0_ref[...], axis=0) + jnp.sum(h5, axis=0)
        out_ref[...] = jnp.dot(tot.reshape(1, 128), wlt_ref[...],
                               preferred_element_type=jnp.float32)

    return pl.pallas_call(
        body,
        out_shape=jax.ShapeDtypeStruct((1, EMB), jnp.float32),
    )(packed_p, b2d, rme, h0, Wlt)


def _score_exp(a):
    heads = a.shape[0]
    ch = a.shape[1]
    eye = jnp.asarray(np.eye(heads, dtype=np.float32))
    bd = (a[:, :, None] * eye[:, None, :]
          ).reshape(heads * ch, heads)
    exp_m = np.zeros((heads, 128), np.float32)
    group = 128 // heads
    for h in range(heads):
        exp_m[h, group * h:group * (h + 1)] = 1.0
    return bd @ jnp.asarray(exp_m)


def _layer_mats(W, a_s, a_d):
    Wt = W.T
    s_exp = Wt @ _score_exp(a_s)
    d_exp = Wt @ _score_exp(a_d)
    M0 = jnp.concatenate([Wt[:, 0:64], s_exp[:, 0:64]], axis=1)
    M1 = jnp.concatenate([Wt[:, 64:128], s_exp[:, 64:128]], axis=1)
    MD = d_exp
    return M0, M1, MD


def kernel(h0, edge_index, W1, a1s, a1d, b1, W2, a2s, a2d, b2,
           W3, a3s, a3d, b3, W4, a4s, a4d, b4, W5, a5s, a5d, b5, Wl):
    loop = jnp.arange(N, dtype=jnp.int32)
    pad = EPAD - E1
    srcp = jnp.concatenate([edge_index[0], loop, jnp.zeros((pad,), jnp.int32)])
    dstp = jnp.concatenate([edge_index[1], loop, jnp.full((pad,), N, jnp.int32)])
    zpc = jnp.zeros((PCH, 128), jnp.float32)
    rme_np = np.zeros((16, 64), np.float32)
    for g in range(4):
        rme_np[g, 16 * g:16 * (g + 1)] = 1.0
    rme = jnp.asarray(rme_np)

    mats = [
        _layer_mats(W1, a1s, a1d),
        _layer_mats(W2, a2s, a2d),
        _layer_mats(W3, a3s, a3d),
        _layer_mats(W4, a4s, a4d),
        _layer_mats(W5, a5s, a5d),
    ]
    biases = [b.reshape(1, 128) for b in (b1, b2, b3, b4, b5)]

    g0, g1, d = _tc_first(h0, *mats[0])
    (packed_p,) = _gat_edges_sc(g0, g1, d, srcp, dstp, zpc)

    for li in range(1, 5):
        g0, g1, d = _tc_mid(packed_p, biases[li - 1], rme, *mats[li])
        (packed_p,) = _gat_edges_sc(g0, g1, d, srcp, dstp, zpc)

    emb = _tc_final(packed_p, biases[4], rme, h0, Wl.T)
    return emb.reshape(EMB)
